# baseline (speedup 1.0000x reference)
import functools

import jax
import jax.numpy as jnp
from jax import lax
from jax.experimental import pallas as pl
from jax.experimental.pallas import tpu as pltpu
from jax.experimental.pallas import tpu_sc as plsc

_NUM_EXAMP = 1000000
_NUM_CLASSES = 32
_BATCH = 16384
_LAMBDA_REG = 3.0
_BETA = 0.7

_NC = 2
_NS = 16
_NW = _NC * _NS
_BPW = _BATCH // _NW
_CHUNK = 128
_K = _BPW // _CHUNK
_PW = 128


@functools.cache
def _sc_kernels():
    mesh = plsc.VectorSubcoreMesh(core_axis_name="c", subcore_axis_name="s")
    scratch = [
        pltpu.VMEM((_K, _CHUNK), jnp.int32),
        pltpu.VMEM((_BPW, _PW), jnp.float32),
        pltpu.SemaphoreType.DMA,
    ]

    @functools.partial(
        pl.kernel,
        out_type=jax.ShapeDtypeStruct((_BATCH, _PW), jnp.float32),
        mesh=mesh,
        scratch_types=scratch,
    )
    def sc_gather(p_ref_hbm, idx_hbm, out_hbm, idx_v, rows_v, sem):
        wid = lax.axis_index("s") * _NC + lax.axis_index("c")
        pltpu.sync_copy(idx_hbm.at[pl.ds(wid * _K, _K)], idx_v)
        handles = []
        for j in range(_K):
            handles.append(
                pltpu.async_copy(
                    p_ref_hbm.at[idx_v.at[j]],
                    rows_v.at[pl.ds(j * _CHUNK, _CHUNK)],
                    sem,
                )
            )
        for h in handles:
            h.wait()
        pltpu.sync_copy(rows_v, out_hbm.at[pl.ds(wid * _BPW, _BPW)])

    @functools.partial(
        pl.kernel,
        out_type=(),
        mesh=mesh,
        scratch_types=scratch,
    )
    def sc_scatter(p_ref_hbm, idx_hbm, rows_hbm, idx_v, rows_v, sem):
        wid = lax.axis_index("s") * _NC + lax.axis_index("c")
        pltpu.sync_copy(idx_hbm.at[pl.ds(wid * _K, _K)], idx_v)
        pltpu.sync_copy(rows_hbm.at[pl.ds(wid * _BPW, _BPW)], rows_v)
        handles = []
        for j in range(_K):
            handles.append(
                pltpu.async_copy(
                    rows_v.at[pl.ds(j * _CHUNK, _CHUNK)],
                    p_ref_hbm.at[idx_v.at[j]],
                    sem,
                )
            )
        for h in handles:
            h.wait()

    return sc_gather, sc_scatter


_BB = 2048
_GRID = _BATCH // _BB

_PB = 2048
_PGRID = -(-_NUM_EXAMP // _PB)


def _padt_body(tt_ref, p_ref):
    x = tt_ref[...]
    eye = (lax.broadcasted_iota(jnp.int32, (_NUM_CLASSES, _NUM_CLASSES), 0)
           == lax.broadcasted_iota(jnp.int32, (_NUM_CLASSES, _NUM_CLASSES), 1)
           ).astype(jnp.float32)
    xt = lax.dot_general(x, eye, (((0,), (0,)), ((), ())),
                         preferred_element_type=jnp.float32)
    p_ref[:, 0:_NUM_CLASSES] = xt
    p_ref[:, _NUM_CLASSES:_PW] = jnp.zeros((_PB, _PW - _NUM_CLASSES), jnp.float32)


_padt = pl.pallas_call(
    _padt_body,
    grid=(_PGRID,),
    in_specs=[pl.BlockSpec((_NUM_CLASSES, _PB), lambda i: (0, i))],
    out_specs=pl.BlockSpec((_PB, _PW), lambda i: (i, 0)),
    out_shape=jax.ShapeDtypeStruct((_NUM_EXAMP, _PW), jnp.float32),
)


def _dense_body(out_ref, lab_ref, grp_ref, newrows_ref, loss_ref, ce_acc, reg_acc):
    i = pl.program_id(0)
    o = out_ref[...]
    m = jnp.max(o, axis=1, keepdims=True)
    e = jnp.exp(o - m)
    s = jnp.sum(e, axis=1, keepdims=True)
    y = jnp.clip(e / s, 0.0001, 1.0 - 0.0001)
    logp = (o - m) - jnp.log(s)
    lab = lab_ref[...]
    onehot = lab == lax.broadcasted_iota(jnp.int32, (_BB, _NUM_CLASSES), 1)
    ce_blk = jnp.sum(jnp.where(onehot, logp, 0.0))
    avg = grp_ref[:, 0:_NUM_CLASSES]
    dot = jnp.sum(avg * y, axis=1, keepdims=True)
    reg_blk = jnp.sum(jnp.log(1.0 - dot))
    newrows_ref[:, 0:_NUM_CLASSES] = _BETA * avg + (1.0 - _BETA) * y
    newrows_ref[:, _NUM_CLASSES:_PW] = jnp.zeros((_BB, _PW - _NUM_CLASSES),
                                                 jnp.float32)

    @pl.when(i == 0)
    def _():
        ce_acc[0, 0] = 0.0
        reg_acc[0, 0] = 0.0

    ce_acc[0, 0] += ce_blk
    reg_acc[0, 0] += reg_blk
    inv_b = 1.0 / _BATCH
    loss_ref[0, 0] = -ce_acc[0, 0] * inv_b + _LAMBDA_REG * reg_acc[0, 0] * inv_b


_dense = pl.pallas_call(
    _dense_body,
    grid=(_GRID,),
    in_specs=[
        pl.BlockSpec((_BB, _NUM_CLASSES), lambda i: (i, 0)),
        pl.BlockSpec((_BB, 1), lambda i: (i, 0)),
        pl.BlockSpec((_BB, _PW), lambda i: (i, 0)),
    ],
    out_specs=[
        pl.BlockSpec((_BB, _PW), lambda i: (i, 0)),
        pl.BlockSpec(memory_space=pltpu.SMEM),
    ],
    out_shape=[
        jax.ShapeDtypeStruct((_BATCH, _PW), jnp.float32),
        jax.ShapeDtypeStruct((1, 1), jnp.float32),
    ],
    scratch_shapes=[
        pltpu.SMEM((1, 1), jnp.float32),
        pltpu.SMEM((1, 1), jnp.float32),
    ],
)


def kernel(index, output, label, target):
    sc_gather, sc_scatter = _sc_kernels()
    idx2 = index.astype(jnp.int32).reshape(_BATCH // _CHUNK, _CHUNK)
    p = _padt(target.T)
    p_ref = jax.new_ref(p)
    y_grp = sc_gather(p_ref, idx2)
    new_rows, loss2 = _dense(output, label.astype(jnp.int32).reshape(_BATCH, 1),
                             y_grp)
    sc_scatter(p_ref, idx2, new_rows)
    new_target = jax.freeze(p_ref)[:, 0:_NUM_CLASSES]
    return loss2[0, 0], new_target


# speedup vs baseline: 1.4669x; 1.4669x over previous
import functools

import jax
import jax.numpy as jnp
from jax import lax
from jax.experimental import pallas as pl
from jax.experimental.pallas import tpu as pltpu
from jax.experimental.pallas import tpu_sc as plsc

_NUM_EXAMP = 1000000
_NUM_CLASSES = 32
_BATCH = 16384
_LAMBDA_REG = 3.0
_BETA = 0.7

_NC = 2
_NS = 16
_NW = _NC * _NS
_BPW = _BATCH // _NW
_CHUNK = 128
_K = _BPW // _CHUNK
_PW = 128


@functools.cache
def _sc_kernels():
    mesh = plsc.VectorSubcoreMesh(core_axis_name="c", subcore_axis_name="s")
    scratch = [
        pltpu.VMEM((_K, _CHUNK), jnp.int32),
        pltpu.VMEM((_BPW, _PW), jnp.float32),
        pltpu.SemaphoreType.DMA,
    ]

    @functools.partial(
        pl.kernel,
        out_type=jax.ShapeDtypeStruct((_BATCH, _PW), jnp.float32),
        mesh=mesh,
        scratch_types=scratch,
    )
    def sc_gather(p_ref_hbm, idx_hbm, out_hbm, idx_v, rows_v, sem):
        wid = lax.axis_index("s") * _NC + lax.axis_index("c")
        pltpu.sync_copy(idx_hbm.at[pl.ds(wid * _K, _K)], idx_v)
        handles = []
        for j in range(_K):
            handles.append(
                pltpu.async_copy(
                    p_ref_hbm.at[idx_v.at[j]],
                    rows_v.at[pl.ds(j * _CHUNK, _CHUNK)],
                    sem,
                )
            )
        for h in handles:
            h.wait()
        pltpu.sync_copy(rows_v, out_hbm.at[pl.ds(wid * _BPW, _BPW)])

    @functools.partial(
        pl.kernel,
        out_type=(),
        mesh=mesh,
        scratch_types=scratch,
    )
    def sc_scatter(p_ref_hbm, idx_hbm, rows_hbm, idx_v, rows_v, sem):
        wid = lax.axis_index("s") * _NC + lax.axis_index("c")
        pltpu.sync_copy(idx_hbm.at[pl.ds(wid * _K, _K)], idx_v)
        pltpu.sync_copy(rows_hbm.at[pl.ds(wid * _BPW, _BPW)], rows_v)
        handles = []
        for j in range(_K):
            handles.append(
                pltpu.async_copy(
                    rows_v.at[pl.ds(j * _CHUNK, _CHUNK)],
                    p_ref_hbm.at[idx_v.at[j]],
                    sem,
                )
            )
        for h in handles:
            h.wait()

    return sc_gather, sc_scatter


_BB = 2048
_GRID = _BATCH // _BB

_PB = 8192
_PGRID = -(-_NUM_EXAMP // _PB)


def _padt_body(tt_ref, p_ref):
    x = tt_ref[...]
    p_ref[:, 0:_NUM_CLASSES] = x.T
    p_ref[:, _NUM_CLASSES:_PW] = jnp.zeros((_PB, _PW - _NUM_CLASSES), jnp.float32)


_padt = pl.pallas_call(
    _padt_body,
    grid=(_PGRID,),
    in_specs=[pl.BlockSpec((_NUM_CLASSES, _PB), lambda i: (0, i))],
    out_specs=pl.BlockSpec((_PB, _PW), lambda i: (i, 0)),
    out_shape=jax.ShapeDtypeStruct((_NUM_EXAMP, _PW), jnp.float32),
)


def _dense_body(out_ref, lab_ref, grp_ref, newrows_ref, loss_ref, ce_acc, reg_acc):
    i = pl.program_id(0)
    o = out_ref[...]
    m = jnp.max(o, axis=1, keepdims=True)
    e = jnp.exp(o - m)
    s = jnp.sum(e, axis=1, keepdims=True)
    y = jnp.clip(e / s, 0.0001, 1.0 - 0.0001)
    logp = (o - m) - jnp.log(s)
    lab = lab_ref[...]
    onehot = lab == lax.broadcasted_iota(jnp.int32, (_BB, _NUM_CLASSES), 1)
    ce_blk = jnp.sum(jnp.where(onehot, logp, 0.0))
    avg = grp_ref[:, 0:_NUM_CLASSES]
    dot = jnp.sum(avg * y, axis=1, keepdims=True)
    reg_blk = jnp.sum(jnp.log(1.0 - dot))
    newrows_ref[:, 0:_NUM_CLASSES] = _BETA * avg + (1.0 - _BETA) * y
    newrows_ref[:, _NUM_CLASSES:_PW] = jnp.zeros((_BB, _PW - _NUM_CLASSES),
                                                 jnp.float32)

    @pl.when(i == 0)
    def _():
        ce_acc[0, 0] = 0.0
        reg_acc[0, 0] = 0.0

    ce_acc[0, 0] += ce_blk
    reg_acc[0, 0] += reg_blk
    inv_b = 1.0 / _BATCH
    loss_ref[0, 0] = -ce_acc[0, 0] * inv_b + _LAMBDA_REG * reg_acc[0, 0] * inv_b


_dense = pl.pallas_call(
    _dense_body,
    grid=(_GRID,),
    in_specs=[
        pl.BlockSpec((_BB, _NUM_CLASSES), lambda i: (i, 0)),
        pl.BlockSpec((_BB, 1), lambda i: (i, 0)),
        pl.BlockSpec((_BB, _PW), lambda i: (i, 0)),
    ],
    out_specs=[
        pl.BlockSpec((_BB, _PW), lambda i: (i, 0)),
        pl.BlockSpec(memory_space=pltpu.SMEM),
    ],
    out_shape=[
        jax.ShapeDtypeStruct((_BATCH, _PW), jnp.float32),
        jax.ShapeDtypeStruct((1, 1), jnp.float32),
    ],
    scratch_shapes=[
        pltpu.SMEM((1, 1), jnp.float32),
        pltpu.SMEM((1, 1), jnp.float32),
    ],
)


def kernel(index, output, label, target):
    sc_gather, sc_scatter = _sc_kernels()
    idx2 = index.astype(jnp.int32).reshape(_BATCH // _CHUNK, _CHUNK)
    p = _padt(target.T)
    p_ref = jax.new_ref(p)
    y_grp = sc_gather(p_ref, idx2)
    new_rows, loss2 = _dense(output, label.astype(jnp.int32).reshape(_BATCH, 1),
                             y_grp)
    sc_scatter(p_ref, idx2, new_rows)
    new_target = jax.freeze(p_ref)[:, 0:_NUM_CLASSES]
    return loss2[0, 0], new_target


# speedup vs baseline: 1.5912x; 1.0848x over previous
import functools

import jax
import jax.numpy as jnp
from jax import lax
from jax.experimental import pallas as pl
from jax.experimental.pallas import tpu as pltpu
from jax.experimental.pallas import tpu_sc as plsc

_NUM_EXAMP = 1000000
_NUM_CLASSES = 32
_BATCH = 16384
_LAMBDA_REG = 3.0
_BETA = 0.7

_NC = 2
_NS = 16
_NW = _NC * _NS
_BPW = _BATCH // _NW
_CHUNK = 128
_K = _BPW // _CHUNK
_PW = 128


@functools.cache
def _sc_kernels():
    mesh = plsc.VectorSubcoreMesh(core_axis_name="c", subcore_axis_name="s")
    scratch = [
        pltpu.VMEM((_K, _CHUNK), jnp.int32),
        pltpu.VMEM((_BPW, _PW), jnp.float32),
        pltpu.SemaphoreType.DMA,
    ]

    @functools.partial(
        pl.kernel,
        out_type=jax.ShapeDtypeStruct((_BATCH, _PW), jnp.float32),
        mesh=mesh,
        scratch_types=scratch,
    )
    def sc_gather(p_ref_hbm, idx_hbm, out_hbm, idx_v, rows_v, sem):
        wid = lax.axis_index("s") * _NC + lax.axis_index("c")
        pltpu.sync_copy(idx_hbm.at[pl.ds(wid * _K, _K)], idx_v)
        handles = []
        for j in range(_K):
            handles.append(
                pltpu.async_copy(
                    p_ref_hbm.at[idx_v.at[j]],
                    rows_v.at[pl.ds(j * _CHUNK, _CHUNK)],
                    sem,
                )
            )
        for h in handles:
            h.wait()
        pltpu.sync_copy(rows_v, out_hbm.at[pl.ds(wid * _BPW, _BPW)])

    @functools.partial(
        pl.kernel,
        out_type=(),
        mesh=mesh,
        scratch_types=scratch,
    )
    def sc_scatter(p_ref_hbm, idx_hbm, rows_hbm, idx_v, rows_v, sem):
        wid = lax.axis_index("s") * _NC + lax.axis_index("c")
        pltpu.sync_copy(idx_hbm.at[pl.ds(wid * _K, _K)], idx_v)
        pltpu.sync_copy(rows_hbm.at[pl.ds(wid * _BPW, _BPW)], rows_v)
        handles = []
        for j in range(_K):
            handles.append(
                pltpu.async_copy(
                    rows_v.at[pl.ds(j * _CHUNK, _CHUNK)],
                    p_ref_hbm.at[idx_v.at[j]],
                    sem,
                )
            )
        for h in handles:
            h.wait()

    return sc_gather, sc_scatter


_BB = 2048
_GRID = _BATCH // _BB

_PB = 32768
_PGRID = -(-_NUM_EXAMP // _PB)


def _padt_body(tt_ref, p_ref):
    x = tt_ref[...]
    p_ref[:, 0:_NUM_CLASSES] = x.T
    p_ref[:, _NUM_CLASSES:_PW] = jnp.zeros((_PB, _PW - _NUM_CLASSES), jnp.float32)


_padt = pl.pallas_call(
    _padt_body,
    grid=(_PGRID,),
    in_specs=[pl.BlockSpec((_NUM_CLASSES, _PB), lambda i: (0, i))],
    out_specs=pl.BlockSpec((_PB, _PW), lambda i: (i, 0)),
    out_shape=jax.ShapeDtypeStruct((_NUM_EXAMP, _PW), jnp.float32),
)


def _dense_body(out_ref, lab_ref, grp_ref, newrows_ref, loss_ref, ce_acc, reg_acc):
    i = pl.program_id(0)
    o = out_ref[...]
    m = jnp.max(o, axis=1, keepdims=True)
    e = jnp.exp(o - m)
    s = jnp.sum(e, axis=1, keepdims=True)
    y = jnp.clip(e / s, 0.0001, 1.0 - 0.0001)
    logp = (o - m) - jnp.log(s)
    lab = lab_ref[...]
    onehot = lab == lax.broadcasted_iota(jnp.int32, (_BB, _NUM_CLASSES), 1)
    ce_blk = jnp.sum(jnp.where(onehot, logp, 0.0))
    avg = grp_ref[:, 0:_NUM_CLASSES]
    dot = jnp.sum(avg * y, axis=1, keepdims=True)
    reg_blk = jnp.sum(jnp.log(1.0 - dot))
    newrows_ref[:, 0:_NUM_CLASSES] = _BETA * avg + (1.0 - _BETA) * y
    newrows_ref[:, _NUM_CLASSES:_PW] = jnp.zeros((_BB, _PW - _NUM_CLASSES),
                                                 jnp.float32)

    @pl.when(i == 0)
    def _():
        ce_acc[0, 0] = 0.0
        reg_acc[0, 0] = 0.0

    ce_acc[0, 0] += ce_blk
    reg_acc[0, 0] += reg_blk
    inv_b = 1.0 / _BATCH
    loss_ref[0, 0] = -ce_acc[0, 0] * inv_b + _LAMBDA_REG * reg_acc[0, 0] * inv_b


_dense = pl.pallas_call(
    _dense_body,
    grid=(_GRID,),
    in_specs=[
        pl.BlockSpec((_BB, _NUM_CLASSES), lambda i: (i, 0)),
        pl.BlockSpec((_BB, 1), lambda i: (i, 0)),
        pl.BlockSpec((_BB, _PW), lambda i: (i, 0)),
    ],
    out_specs=[
        pl.BlockSpec((_BB, _PW), lambda i: (i, 0)),
        pl.BlockSpec(memory_space=pltpu.SMEM),
    ],
    out_shape=[
        jax.ShapeDtypeStruct((_BATCH, _PW), jnp.float32),
        jax.ShapeDtypeStruct((1, 1), jnp.float32),
    ],
    scratch_shapes=[
        pltpu.SMEM((1, 1), jnp.float32),
        pltpu.SMEM((1, 1), jnp.float32),
    ],
)


def kernel(index, output, label, target):
    sc_gather, sc_scatter = _sc_kernels()
    idx2 = index.astype(jnp.int32).reshape(_BATCH // _CHUNK, _CHUNK)
    p = _padt(target.T)
    p_ref = jax.new_ref(p)
    y_grp = sc_gather(p_ref, idx2)
    new_rows, loss2 = _dense(output, label.astype(jnp.int32).reshape(_BATCH, 1),
                             y_grp)
    sc_scatter(p_ref, idx2, new_rows)
    new_target = jax.freeze(p_ref)[:, 0:_NUM_CLASSES]
    return loss2[0, 0], new_target
